# baseline (speedup 1.0000x reference)
import functools

import jax
import jax.numpy as jnp
from jax.experimental import pallas as pl


def _tc_body(w_ref, m_ref, w1_ref, b1_ref, w2_ref, b2_ref,
             p_ref, sel_ref, ll_ref, *, n_cols, n_hidden, rb, rsub, ck):
    n_ck = n_cols // ck
    w1 = w1_ref[...]
    a_bf = (w1[:, 0].reshape(1, n_hidden)
            .astype(jnp.bfloat16).astype(jnp.float32))
    c_bf = (w1[:, 1].reshape(1, n_hidden)
            .astype(jnp.bfloat16).astype(jnp.float32))
    b1v = b1_ref[...]
    w2v = w2_ref[...]
    b2s = b2_ref[...][0:1, 0:1]

    for rg in range(rb // rsub):
        r0 = rg * rsub
        rows = slice(r0, r0 + rsub)

        def sum_step(c, acc):
            wt = w_ref[rows, pl.ds(c * ck, ck)]
            return acc + jnp.sum(wt, axis=1, keepdims=True)

        rsum = jax.lax.fori_loop(0, n_ck, sum_step,
                                 jnp.zeros((rsub, 1), jnp.float32))
        mean = rsum * (1.0 / n_cols)
        mean_bf = mean.astype(jnp.bfloat16).astype(jnp.float32)

        def pi_step(c, rmax):
            wt = w_ref[rows, pl.ds(c * ck, ck)]
            mt = m_ref[rows, pl.ds(c * ck, ck)]
            wbf = wt.astype(jnp.bfloat16).astype(jnp.float32)
            acc = jnp.zeros((rsub, ck), jnp.float32)
            for h in range(n_hidden):
                u = wbf * a_bf[0:1, h:h + 1] + mean_bf * c_bf[0:1, h:h + 1]
                hh = jnp.maximum(u + b1v[0:1, h:h + 1], 0.0)
                hh = hh.astype(jnp.bfloat16).astype(jnp.float32)
                acc = acc + hh * w2v[0:1, h:h + 1]
            pi = acc + b2s
            pi = jnp.where(mt != 0, -1e6, pi)
            p_ref[rows, pl.ds(c * ck, ck)] = pi
            return jnp.maximum(rmax, jnp.max(pi, axis=1, keepdims=True))

        rmax = jax.lax.fori_loop(0, n_ck, pi_step,
                                 jnp.full((rsub, 1), -jnp.inf, jnp.float32))

        def exp_step(c, ssum):
            pi = p_ref[rows, pl.ds(c * ck, ck)]
            e = jnp.exp(pi - rmax)
            p_ref[rows, pl.ds(c * ck, ck)] = e
            return ssum + jnp.sum(e, axis=1, keepdims=True)

        ssum = jax.lax.fori_loop(0, n_ck, exp_step,
                                 jnp.zeros((rsub, 1), jnp.float32))
        pmax = 1.0 / ssum

        def norm_step(c, selmin):
            e = p_ref[rows, pl.ds(c * ck, ck)]
            p0 = e / ssum
            p_ref[rows, pl.ds(c * ck, ck)] = p0 + 1e-6
            idx = jax.lax.broadcasted_iota(jnp.int32, (rsub, ck), 1) + c * ck
            cand = jnp.min(jnp.where(p0 == pmax, idx, n_cols),
                           axis=1, keepdims=True)
            return jnp.minimum(selmin, cand)

        selmin = jax.lax.fori_loop(0, n_ck, norm_step,
                                   jnp.full((rsub, 1), n_cols, jnp.int32))

        sel_ref[0, 0, r0:r0 + rsub] = selmin[:, 0]
        ll_ref[0, 0, r0:r0 + rsub] = jnp.log(pmax[:, 0] + 1e-6)


def kernel(w, mask, W1, b1, W2, b2):
    n_rows, n_cols = w.shape
    n_hidden = W1.shape[0]
    rb = 32 if n_rows % 32 == 0 else n_rows
    g = n_rows // rb

    mask8 = mask.astype(jnp.int8)
    b1r = b1.reshape(1, n_hidden)
    w2r = W2.reshape(1, n_hidden)
    b2r = b2.reshape(1, 1)

    body = functools.partial(_tc_body, n_cols=n_cols, n_hidden=n_hidden,
                             rb=rb, rsub=8, ck=min(1024, n_cols))
    p, sel3, ll3 = pl.pallas_call(
        body,
        grid=(g,),
        in_specs=[
            pl.BlockSpec((rb, n_cols), lambda i: (i, 0)),
            pl.BlockSpec((rb, n_cols), lambda i: (i, 0)),
            pl.BlockSpec((n_hidden, 2), lambda i: (0, 0)),
            pl.BlockSpec((1, n_hidden), lambda i: (0, 0)),
            pl.BlockSpec((1, n_hidden), lambda i: (0, 0)),
            pl.BlockSpec((1, 1), lambda i: (0, 0)),
        ],
        out_specs=[
            pl.BlockSpec((rb, n_cols), lambda i: (i, 0)),
            pl.BlockSpec((1, 1, rb), lambda i: (i, 0, 0)),
            pl.BlockSpec((1, 1, rb), lambda i: (i, 0, 0)),
        ],
        out_shape=[
            jax.ShapeDtypeStruct((n_rows, n_cols), jnp.float32),
            jax.ShapeDtypeStruct((g, 1, rb), jnp.int32),
            jax.ShapeDtypeStruct((g, 1, rb), jnp.float32),
        ],
    )(w, mask8, W1, b1r, w2r, b2r)

    return sel3.reshape(n_rows), p, ll3.reshape(n_rows)
